# Initial kernel scaffold; baseline (speedup 1.0000x reference)
import jax
import jax.numpy as jnp
from jax.experimental import pallas as pl


def kernel(x, edge_index, boundary_mask, W1, b1, W2, b2):
    raise NotImplementedError("write your pallas kernel here")


# speedup vs baseline: 13.3227x; 13.3227x over previous
import functools

import jax
import jax.numpy as jnp
from jax import lax
from jax.experimental import pallas as pl
from jax.experimental.pallas import tpu as pltpu
from jax.experimental.pallas import tpu_sc as plsc

NC = 2
NS = 16
NW = NC * NS
K = 80
L = 16


def _seg_sum_rows(n_pad, d, e):
    epw = e // NW
    nchunk = epw // K
    stripe = n_pad // NS

    mesh = plsc.VectorSubcoreMesh(
        core_axis_name="c", subcore_axis_name="s", num_cores=NC, num_subcores=NS)

    @functools.partial(
        pl.kernel,
        mesh=mesh,
        out_type=jax.ShapeDtypeStruct((NC, n_pad, d), jnp.float32),
        scratch_types=[
            pltpu.VMEM((K,), jnp.int32),
            pltpu.VMEM((K,), jnp.int32),
            pltpu.VMEM((K, d), jnp.float32),
            pltpu.VMEM_SHARED((n_pad, d), jnp.float32),
            pltpu.SemaphoreType.DMA,
        ],
    )
    def seg_kernel(h_hbm, src_hbm, dst_hbm, out_hbm, srcv, dstv, rows_v, acc_sh, sem):
        c = lax.axis_index("c")
        s = lax.axis_index("s")
        wid = c * NS + s
        sbase = s * stripe

        zero = jnp.zeros((L,), jnp.float32)

        def zbody(i, _):
            for j in range(d // L):
                rows_v[i, pl.ds(j * L, L)] = zero
            return 0

        lax.fori_loop(0, K, zbody, 0)
        for t in range(stripe // K):
            pltpu.sync_copy(rows_v, acc_sh.at[pl.ds(sbase + t * K, K)])
        plsc.subcore_barrier()

        ebase = wid * epw

        def body(j, _):
            off = pl.multiple_of(ebase + j * K, 8)
            pltpu.sync_copy(src_hbm.at[pl.ds(off, K)], srcv)
            pltpu.sync_copy(dst_hbm.at[pl.ds(off, K)], dstv)
            pltpu.async_copy(h_hbm.at[srcv], rows_v, sem).wait()
            pltpu.sync_copy(rows_v, acc_sh.at[dstv], add=True)
            return 0

        lax.fori_loop(0, nchunk, body, 0)
        plsc.subcore_barrier()

        for t in range(stripe // K):
            pltpu.sync_copy(acc_sh.at[pl.ds(sbase + t * K, K)], rows_v)
            pltpu.sync_copy(rows_v, out_hbm.at[c, pl.ds(sbase + t * K, K)])

    return seg_kernel


def _deg_count(n_pad, e):
    epw = e // NW
    nchunk = epw // K
    stripe = n_pad // NS

    mesh = plsc.VectorSubcoreMesh(
        core_axis_name="c", subcore_axis_name="s", num_cores=NC, num_subcores=NS)

    @functools.partial(
        pl.kernel,
        mesh=mesh,
        out_type=jax.ShapeDtypeStruct((NC, n_pad), jnp.float32),
        scratch_types=[
            pltpu.VMEM((K,), jnp.int32),
            pltpu.VMEM((K,), jnp.float32),
            pltpu.VMEM((stripe,), jnp.float32),
            pltpu.VMEM_SHARED((n_pad,), jnp.float32),
            pltpu.SemaphoreType.DMA,
        ],
    )
    def deg_kernel(dst_hbm, out_hbm, dstv, ones_v, stripe_v, acc_sh, sem):
        c = lax.axis_index("c")
        s = lax.axis_index("s")
        wid = c * NS + s
        sbase = s * stripe

        one = jnp.ones((L,), jnp.float32)
        zero = jnp.zeros((L,), jnp.float32)
        for i in range(K // L):
            ones_v[pl.ds(i * L, L)] = one

        def zbody(i, _):
            stripe_v[pl.ds(i * L, L)] = zero
            return 0

        lax.fori_loop(0, stripe // L, zbody, 0)
        pltpu.sync_copy(stripe_v, acc_sh.at[pl.ds(sbase, stripe)])
        plsc.subcore_barrier()

        ebase = wid * epw

        def body(j, _):
            off = pl.multiple_of(ebase + j * K, 8)
            pltpu.sync_copy(dst_hbm.at[pl.ds(off, K)], dstv)
            pltpu.sync_copy(ones_v, acc_sh.at[dstv], add=True)
            return 0

        lax.fori_loop(0, nchunk, body, 0)
        plsc.subcore_barrier()

        pltpu.sync_copy(acc_sh.at[pl.ds(sbase, stripe)], stripe_v)
        pltpu.sync_copy(stripe_v, out_hbm.at[c, pl.ds(sbase, stripe)])

    return deg_kernel


def _t1_body(cnt_ref, x_ref, w_ref, dis_ref, hp_ref):
    cnt = cnt_ref[0] + cnt_ref[1]
    dis = lax.rsqrt(cnt + 1.0)
    h = jnp.dot(x_ref[...], w_ref[...], preferred_element_type=jnp.float32)
    dis_ref[...] = dis
    hp_ref[...] = h * dis


def _t2_body(a_ref, hp_ref, dis_ref, w_ref, b_ref, out_ref):
    z = (a_ref[0] + a_ref[1] + hp_ref[...]) * dis_ref[...] + b_ref[...]
    h = jnp.maximum(z, 0.0)
    out_ref[...] = jnp.dot(h, w_ref[...], preferred_element_type=jnp.float32) * dis_ref[...]


def _t3_body(a_ref, hp_ref, dis_ref, b_ref, m_ref, out_ref):
    z = (a_ref[0] + a_ref[1] + hp_ref[...]) * dis_ref[...] + b_ref[...]
    out_ref[...] = jnp.where(m_ref[...] > 0.0, 0.0, z)


def kernel(x, edge_index, boundary_mask, W1, b1, W2, b2):
    n, d = x.shape
    e = edge_index.shape[1]
    n_pad = ((n + (NS * K) - 1) // (NS * K)) * (NS * K)
    bn = 2000
    grid = n // bn

    src = edge_index[0].astype(jnp.int32)
    dst = edge_index[1].astype(jnp.int32)
    mask = boundary_mask.astype(jnp.float32).reshape(n, 1)

    cnt = _deg_count(n_pad, e)(dst).reshape(NC, n_pad, 1)

    row_spec = pl.BlockSpec((bn, d), lambda i: (i, 0))
    col1_spec = pl.BlockSpec((bn, 1), lambda i: (i, 0))
    pair_spec = pl.BlockSpec((NC, bn, d), lambda i: (0, i, 0))
    pair1_spec = pl.BlockSpec((NC, bn, 1), lambda i: (0, i, 0))
    w_spec = pl.BlockSpec((d, d), lambda i: (0, 0))
    b_spec = pl.BlockSpec((1, d), lambda i: (0, 0))

    dis, h1p = pl.pallas_call(
        _t1_body,
        grid=(grid,),
        in_specs=[pair1_spec, row_spec, w_spec],
        out_specs=[col1_spec, row_spec],
        out_shape=[
            jax.ShapeDtypeStruct((n, 1), jnp.float32),
            jax.ShapeDtypeStruct((n, d), jnp.float32),
        ],
    )(cnt, x, W1)

    seg = _seg_sum_rows(n_pad, d, e)
    a1 = seg(h1p, src, dst)

    h2p = pl.pallas_call(
        _t2_body,
        grid=(grid,),
        in_specs=[pair_spec, row_spec, col1_spec, w_spec, b_spec],
        out_specs=row_spec,
        out_shape=jax.ShapeDtypeStruct((n, d), jnp.float32),
    )(a1, h1p, dis, W2, b1.reshape(1, d))

    a2 = seg(h2p, src, dst)

    out = pl.pallas_call(
        _t3_body,
        grid=(grid,),
        in_specs=[pair_spec, row_spec, col1_spec, b_spec, col1_spec],
        out_specs=row_spec,
        out_shape=jax.ShapeDtypeStruct((n, d), jnp.float32),
    )(a2, h2p, dis, b2.reshape(1, d), mask)

    return out


# speedup vs baseline: 20.3285x; 1.5259x over previous
import functools

import jax
import jax.numpy as jnp
from jax import lax
from jax.experimental import pallas as pl
from jax.experimental.pallas import tpu as pltpu
from jax.experimental.pallas import tpu_sc as plsc

NC = 2
NS = 16
NW = NC * NS
K = 80
L = 16
NBUF = 5


def _seg_sum_rows(n_pad, d, e):
    epw = e // NW
    nchunk = epw // K
    nrounds = nchunk // NBUF
    stripe = n_pad // NS

    mesh = plsc.VectorSubcoreMesh(
        core_axis_name="c", subcore_axis_name="s", num_cores=NC, num_subcores=NS)

    @functools.partial(
        pl.kernel,
        mesh=mesh,
        out_type=jax.ShapeDtypeStruct((NC, n_pad, d), jnp.float32),
        scratch_types=[
            pltpu.VMEM((nchunk, K), jnp.int32),
            pltpu.VMEM((nchunk, K), jnp.int32),
            [pltpu.VMEM((K, d), jnp.float32)] * NBUF,
            pltpu.VMEM_SHARED((n_pad, d), jnp.float32),
            pltpu.SemaphoreType.DMA,
        ],
    )
    def seg_kernel(h_hbm, src_hbm, dst_hbm, out_hbm, srcv, dstv, rows_bufs, acc_sh,
                   sem):
        c = lax.axis_index("c")
        s = lax.axis_index("s")
        wid = c * NS + s
        sbase = s * stripe

        idx_src = pltpu.async_copy(src_hbm.at[wid], srcv, sem)
        idx_dst = pltpu.async_copy(dst_hbm.at[wid], dstv, sem)

        zero = jnp.zeros((L,), jnp.float32)

        def zbody(i, _):
            for j in range(d // L):
                rows_bufs[0][i, pl.ds(j * L, L)] = zero
            return 0

        lax.fori_loop(0, K, zbody, 0)
        for t in range(stripe // K):
            pltpu.sync_copy(rows_bufs[0], acc_sh.at[pl.ds(sbase + t * K, K)])
        idx_src.wait()
        idx_dst.wait()
        plsc.subcore_barrier()

        def body(j, _):
            pltpu.async_copy(h_hbm.at[srcv.at[j]], rows_bufs[0], sem).wait()
            pltpu.async_copy(
                rows_bufs[0], acc_sh.at[dstv.at[j]], sem, add=True).wait()
            return 0

        lax.fori_loop(0, nchunk, body, 0)
        plsc.subcore_barrier()

        for t in range(stripe // K):
            pltpu.sync_copy(acc_sh.at[pl.ds(sbase + t * K, K)], rows_bufs[0])
            pltpu.sync_copy(rows_bufs[0], out_hbm.at[c, pl.ds(sbase + t * K, K)])

    return seg_kernel


def _deg_count(n_pad, e):
    epw = e // NW
    nchunk = epw // K
    stripe = n_pad // NS

    mesh = plsc.VectorSubcoreMesh(
        core_axis_name="c", subcore_axis_name="s", num_cores=NC, num_subcores=NS)

    @functools.partial(
        pl.kernel,
        mesh=mesh,
        out_type=jax.ShapeDtypeStruct((NC, n_pad), jnp.float32),
        scratch_types=[
            pltpu.VMEM((nchunk, K), jnp.int32),
            pltpu.VMEM((K,), jnp.float32),
            pltpu.VMEM((stripe,), jnp.float32),
            pltpu.VMEM_SHARED((n_pad,), jnp.float32),
            pltpu.SemaphoreType.DMA,
        ],
    )
    def deg_kernel(dst_hbm, out_hbm, dstv, ones_v, stripe_v, acc_sh, sem):
        c = lax.axis_index("c")
        s = lax.axis_index("s")
        wid = c * NS + s
        sbase = s * stripe

        idx_dst = pltpu.async_copy(dst_hbm.at[wid], dstv, sem)
        one = jnp.ones((L,), jnp.float32)
        zero = jnp.zeros((L,), jnp.float32)
        for i in range(K // L):
            ones_v[pl.ds(i * L, L)] = one

        def zbody(i, _):
            stripe_v[pl.ds(i * L, L)] = zero
            return 0

        lax.fori_loop(0, stripe // L, zbody, 0)
        pltpu.sync_copy(stripe_v, acc_sh.at[pl.ds(sbase, stripe)])
        idx_dst.wait()
        plsc.subcore_barrier()

        def body(j, _):
            pltpu.sync_copy(ones_v, acc_sh.at[dstv.at[j]], add=True)
            return 0

        lax.fori_loop(0, nchunk, body, 0)
        plsc.subcore_barrier()

        pltpu.sync_copy(acc_sh.at[pl.ds(sbase, stripe)], stripe_v)
        pltpu.sync_copy(stripe_v, out_hbm.at[c, pl.ds(sbase, stripe)])

    return deg_kernel


def _t1_body(cnt_ref, x_ref, w_ref, dis_ref, hp_ref):
    cnt = cnt_ref[0] + cnt_ref[1]
    dis = lax.rsqrt(cnt + 1.0)
    h = jnp.dot(x_ref[...], w_ref[...], preferred_element_type=jnp.float32)
    dis_ref[...] = dis
    hp_ref[...] = h * dis


def _t2_body(a_ref, hp_ref, dis_ref, w_ref, b_ref, out_ref):
    z = (a_ref[0] + a_ref[1] + hp_ref[...]) * dis_ref[...] + b_ref[...]
    h = jnp.maximum(z, 0.0)
    out_ref[...] = jnp.dot(h, w_ref[...], preferred_element_type=jnp.float32) * dis_ref[...]


def _t3_body(a_ref, hp_ref, dis_ref, b_ref, m_ref, out_ref):
    z = (a_ref[0] + a_ref[1] + hp_ref[...]) * dis_ref[...] + b_ref[...]
    out_ref[...] = jnp.where(m_ref[...] > 0.0, 0.0, z)


def kernel(x, edge_index, boundary_mask, W1, b1, W2, b2):
    n, d = x.shape
    e = edge_index.shape[1]
    n_pad = ((n + (NS * K) - 1) // (NS * K)) * (NS * K)
    bn = 2000
    grid = n // bn

    epw = e // NW
    nchunk = epw // K
    src = edge_index[0].astype(jnp.int32).reshape(NW, nchunk, K)
    dst = edge_index[1].astype(jnp.int32).reshape(NW, nchunk, K)
    mask = boundary_mask.astype(jnp.float32).reshape(n, 1)

    cnt = _deg_count(n_pad, e)(dst).reshape(NC, n_pad, 1)

    row_spec = pl.BlockSpec((bn, d), lambda i: (i, 0))
    col1_spec = pl.BlockSpec((bn, 1), lambda i: (i, 0))
    pair_spec = pl.BlockSpec((NC, bn, d), lambda i: (0, i, 0))
    pair1_spec = pl.BlockSpec((NC, bn, 1), lambda i: (0, i, 0))
    w_spec = pl.BlockSpec((d, d), lambda i: (0, 0))
    b_spec = pl.BlockSpec((1, d), lambda i: (0, 0))

    dis, h1p = pl.pallas_call(
        _t1_body,
        grid=(grid,),
        in_specs=[pair1_spec, row_spec, w_spec],
        out_specs=[col1_spec, row_spec],
        out_shape=[
            jax.ShapeDtypeStruct((n, 1), jnp.float32),
            jax.ShapeDtypeStruct((n, d), jnp.float32),
        ],
    )(cnt, x, W1)

    seg = _seg_sum_rows(n_pad, d, e)
    a1 = seg(h1p, src, dst)

    h2p = pl.pallas_call(
        _t2_body,
        grid=(grid,),
        in_specs=[pair_spec, row_spec, col1_spec, w_spec, b_spec],
        out_specs=row_spec,
        out_shape=jax.ShapeDtypeStruct((n, d), jnp.float32),
    )(a1, h1p, dis, W2, b1.reshape(1, d))

    a2 = seg(h2p, src, dst)

    out = pl.pallas_call(
        _t3_body,
        grid=(grid,),
        in_specs=[pair_spec, row_spec, col1_spec, b_spec, col1_spec],
        out_specs=row_spec,
        out_shape=jax.ShapeDtypeStruct((n, d), jnp.float32),
    )(a2, h2p, dis, b2.reshape(1, d), mask)

    return out
